# baseline (speedup 1.0000x reference)
import functools

import jax
import jax.numpy as jnp
from jax import lax
from jax.experimental import pallas as pl
from jax.experimental.pallas import tpu as pltpu
from jax.experimental.pallas import tpu_sc as plsc

_N = 10000
_NC = 2
_NS = 16
_NW = _NC * _NS
_SL = 128
_ACC_ROWS = 10240
_RPT = _ACC_ROWS // _NS

_mesh = plsc.VectorSubcoreMesh(core_axis_name="c", subcore_axis_name="s")
_sc_params = pltpu.CompilerParams(use_tc_tiling_on_sc=False)


def _deg_kernel_body(didx_hbm, zeros_hbm, ones_hbm, out_hbm,
                     didx_v, ones_v, acc_sh, sem):
    cid = lax.axis_index("c")
    sid = lax.axis_index("s")
    wid = cid * _NS + sid
    k = didx_hbm.shape[1]

    pltpu.sync_copy(zeros_hbm.at[pl.ds(0, _RPT)], acc_sh.at[pl.ds(sid * _RPT, _RPT)])
    pltpu.sync_copy(ones_hbm, ones_v)
    pltpu.sync_copy(didx_hbm.at[wid], didx_v)
    plsc.subcore_barrier()

    def body(j, carry):
        pltpu.sync_copy(ones_v, acc_sh.at[didx_v.at[j]], add=True)
        return carry

    lax.fori_loop(0, k, body, 0, unroll=False)
    plsc.subcore_barrier()
    pltpu.sync_copy(acc_sh.at[pl.ds(sid * _RPT, _RPT)],
                    out_hbm.at[cid, pl.ds(sid * _RPT, _RPT)])


def _make_deg(k_slices):
    return functools.partial(
        pl.kernel,
        out_type=jax.ShapeDtypeStruct((_NC, _ACC_ROWS), jnp.float32),
        mesh=_mesh,
        scratch_types=[
            pltpu.VMEM((k_slices, _SL), jnp.int32),
            pltpu.VMEM((_SL,), jnp.float32),
            pltpu.VMEM_SHARED((_ACC_ROWS,), jnp.float32),
            pltpu.SemaphoreType.DMA,
        ],
        compiler_params=_sc_params,
    )(_deg_kernel_body)


_NBUF = 4


def _agg_kernel_body(y_hbm, sidx_hbm, didx_hbm, zeros_hbm, out_hbm,
                     sidx_v, didx_v, rows_v, acc_sh, *sems):
    cid = lax.axis_index("c")
    sid = lax.axis_index("s")
    wid = cid * _NS + sid
    k = sidx_hbm.shape[1]
    groups = k // _NBUF

    pltpu.sync_copy(zeros_hbm.at[pl.ds(0, _RPT)], acc_sh.at[pl.ds(sid * _RPT, _RPT)])
    pltpu.sync_copy(sidx_hbm.at[wid], sidx_v)
    pltpu.sync_copy(didx_hbm.at[wid], didx_v)
    plsc.subcore_barrier()

    for b in range(_NBUF):
        pltpu.async_copy(y_hbm.at[sidx_v.at[b]], rows_v.at[b], sems[b])

    def body(g, carry):
        for b in range(_NBUF):
            s = g * _NBUF + b
            pltpu.make_async_copy(y_hbm.at[sidx_v.at[s]], rows_v.at[b], sems[b]).wait()
            pltpu.async_copy(rows_v.at[b], acc_sh.at[didx_v.at[s]], sems[b], add=True).wait()
            pltpu.async_copy(y_hbm.at[sidx_v.at[s + _NBUF]], rows_v.at[b], sems[b])
        return carry

    lax.fori_loop(0, groups - 1, body, 0, unroll=False)
    for b in range(_NBUF):
        s = (groups - 1) * _NBUF + b
        pltpu.make_async_copy(y_hbm.at[sidx_v.at[s]], rows_v.at[b], sems[b]).wait()
        pltpu.async_copy(rows_v.at[b], acc_sh.at[didx_v.at[s]], sems[b], add=True).wait()

    plsc.subcore_barrier()
    pltpu.sync_copy(acc_sh.at[pl.ds(sid * _RPT, _RPT)],
                    out_hbm.at[cid, pl.ds(sid * _RPT, _RPT)])


def _make_agg(feat, k_slices):
    return functools.partial(
        pl.kernel,
        out_type=jax.ShapeDtypeStruct((_NC, _ACC_ROWS, feat), jnp.float32),
        mesh=_mesh,
        scratch_types=[
            pltpu.VMEM((k_slices, _SL), jnp.int32),
            pltpu.VMEM((k_slices, _SL), jnp.int32),
            pltpu.VMEM((_NBUF, _SL, feat), jnp.float32),
            pltpu.VMEM_SHARED((_ACC_ROWS, feat), jnp.float32),
        ] + [pltpu.SemaphoreType.DMA] * _NBUF,
        compiler_params=_sc_params,
    )(_agg_kernel_body)


def _tc_scale1_body(x_ref, w_ref, degp_ref, y_ref, dis_ref):
    deg = degp_ref[0, : _N, :] + degp_ref[1, : _N, :] + 1.0
    dis = lax.rsqrt(deg)
    xw = jnp.dot(x_ref[...], w_ref[...], preferred_element_type=jnp.float32)
    y_ref[...] = xw * dis
    dis_ref[...] = dis


def _tc_mid_body(p_ref, y1_ref, dis_ref, b1_ref, w2_ref, y2_ref):
    dis = dis_ref[...]
    agg = p_ref[0, : _N, :] + p_ref[1, : _N, :] + y1_ref[...]
    h = jnp.maximum(agg * dis + b1_ref[...], 0.0)
    xw2 = jnp.dot(h, w2_ref[...], preferred_element_type=jnp.float32)
    y2_ref[...] = xw2 * dis


def _tc_final_body(p_ref, y2_ref, dis_ref, b2_ref, o_ref):
    agg = p_ref[0, : _N, :] + p_ref[1, : _N, :] + y2_ref[...]
    z = agg * dis_ref[...] + b2_ref[...]
    m = jnp.max(z, axis=1, keepdims=True)
    lse = jnp.log(jnp.sum(jnp.exp(z - m), axis=1, keepdims=True)) + m
    o_ref[...] = z - lse


def kernel(x, edge_index, W1, b1, W2, b2):
    n, d_in = x.shape
    d_hid = W1.shape[1]
    d_out = W2.shape[1]
    e = edge_index.shape[1]

    src = edge_index[0].astype(jnp.int32)
    dst = edge_index[1].astype(jnp.int32)
    k_slices = -(-e // (_NW * _SL))
    k_slices = -(-k_slices // _NBUF) * _NBUF
    e_pad = _NW * _SL * k_slices
    src_p = jnp.concatenate([src, jnp.zeros((e_pad - e,), jnp.int32)])
    dst_p = jnp.concatenate([dst, jnp.full((e_pad - e,), _N, jnp.int32)])
    src3 = src_p.reshape(_NW, k_slices, _SL)
    dst3 = dst_p.reshape(_NW, k_slices, _SL)
    zeros_hbm = jnp.zeros((_RPT, max(d_hid, d_out)), jnp.float32)

    degp = _make_deg(k_slices)(dst3, zeros_hbm[:, 0], jnp.ones((_SL,), jnp.float32))

    y1, dis = pl.pallas_call(
        _tc_scale1_body,
        out_shape=(
            jax.ShapeDtypeStruct((n, d_hid), jnp.float32),
            jax.ShapeDtypeStruct((n, 1), jnp.float32),
        ),
    )(x, W1, degp.reshape(_NC, _ACC_ROWS, 1))

    p1 = _make_agg(d_hid, k_slices)(y1, src3, dst3, zeros_hbm[:, :d_hid])

    y2 = pl.pallas_call(
        _tc_mid_body,
        out_shape=jax.ShapeDtypeStruct((n, d_out), jnp.float32),
    )(p1, y1, dis, b1.reshape(1, d_hid), W2)

    p2 = _make_agg(d_out, k_slices)(y2, src3, dst3, zeros_hbm[:, :d_out])

    out = pl.pallas_call(
        _tc_final_body,
        out_shape=jax.ShapeDtypeStruct((n, d_out), jnp.float32),
    )(p2, y2, dis, b2.reshape(1, d_out))
    return out


# speedup vs baseline: 1.4580x; 1.4580x over previous
import functools

import jax
import jax.numpy as jnp
from jax import lax
from jax.experimental import pallas as pl
from jax.experimental.pallas import tpu as pltpu
from jax.experimental.pallas import tpu_sc as plsc

_N = 10000
_NC = 2
_NS = 16
_NW = _NC * _NS
_SL = 128
_ACC_ROWS = 10240
_RPT = _ACC_ROWS // _NS

_mesh = plsc.VectorSubcoreMesh(core_axis_name="c", subcore_axis_name="s")
_sc_params = pltpu.CompilerParams(use_tc_tiling_on_sc=False)


def _deg_kernel_body(didx_hbm, zeros_hbm, ones_hbm, out_hbm,
                     didx_v, ones_v, acc_sh, sem):
    cid = lax.axis_index("c")
    sid = lax.axis_index("s")
    wid = cid * _NS + sid
    k = didx_hbm.shape[1]

    pltpu.sync_copy(zeros_hbm.at[pl.ds(0, _RPT)], acc_sh.at[pl.ds(sid * _RPT, _RPT)])
    pltpu.sync_copy(ones_hbm, ones_v)
    pltpu.sync_copy(didx_hbm.at[wid], didx_v)
    plsc.subcore_barrier()

    def body(j, carry):
        pltpu.sync_copy(ones_v, acc_sh.at[didx_v.at[j]], add=True)
        return carry

    lax.fori_loop(0, k, body, 0, unroll=False)
    plsc.subcore_barrier()
    pltpu.sync_copy(acc_sh.at[pl.ds(sid * _RPT, _RPT)],
                    out_hbm.at[cid, pl.ds(sid * _RPT, _RPT)])


def _make_deg(k_slices):
    return functools.partial(
        pl.kernel,
        out_type=jax.ShapeDtypeStruct((_NC, _ACC_ROWS), jnp.float32),
        mesh=_mesh,
        scratch_types=[
            pltpu.VMEM((k_slices, _SL), jnp.int32),
            pltpu.VMEM((_SL,), jnp.float32),
            pltpu.VMEM_SHARED((_ACC_ROWS,), jnp.float32),
            pltpu.SemaphoreType.DMA,
        ],
        compiler_params=_sc_params,
    )(_deg_kernel_body)


_NBUF = 4


def _agg_kernel_body(y_hbm, sidx_hbm, didx_hbm, zeros_hbm, out_hbm,
                     sidx_v, didx_v, rows_v, acc_sh, *sems):
    cid = lax.axis_index("c")
    sid = lax.axis_index("s")
    wid = cid * _NS + sid
    k = sidx_hbm.shape[1]
    groups = k // _NBUF

    pltpu.sync_copy(zeros_hbm.at[pl.ds(0, _RPT)], acc_sh.at[pl.ds(sid * _RPT, _RPT)])
    pltpu.sync_copy(sidx_hbm.at[wid], sidx_v)
    pltpu.sync_copy(didx_hbm.at[wid], didx_v)
    plsc.subcore_barrier()

    for b in range(_NBUF):
        pltpu.async_copy(y_hbm.at[sidx_v.at[b]], rows_v.at[b], sems[b])

    def body(g, carry):
        for b in range(_NBUF):
            s = g * _NBUF + b
            pltpu.make_async_copy(y_hbm.at[sidx_v.at[s]], rows_v.at[b], sems[b]).wait()
            pltpu.async_copy(rows_v.at[b], acc_sh.at[didx_v.at[s]], sems[b], add=True).wait()
            pltpu.async_copy(y_hbm.at[sidx_v.at[s + _NBUF]], rows_v.at[b], sems[b])
        return carry

    lax.fori_loop(0, groups - 1, body, 0, unroll=False)
    for b in range(_NBUF):
        s = (groups - 1) * _NBUF + b
        pltpu.make_async_copy(y_hbm.at[sidx_v.at[s]], rows_v.at[b], sems[b]).wait()
        pltpu.async_copy(rows_v.at[b], acc_sh.at[didx_v.at[s]], sems[b], add=True).wait()

    plsc.subcore_barrier()
    pltpu.sync_copy(acc_sh.at[pl.ds(sid * _RPT, _RPT)],
                    out_hbm.at[cid, pl.ds(sid * _RPT, _RPT)])


def _make_agg(feat, k_slices):
    return functools.partial(
        pl.kernel,
        out_type=jax.ShapeDtypeStruct((_NC, _ACC_ROWS, feat), jnp.float32),
        mesh=_mesh,
        scratch_types=[
            pltpu.VMEM((k_slices, _SL), jnp.int32),
            pltpu.VMEM((k_slices, _SL), jnp.int32),
            pltpu.VMEM((_NBUF, _SL, feat), jnp.float32),
            pltpu.VMEM_SHARED((_ACC_ROWS, feat), jnp.float32),
        ] + [pltpu.SemaphoreType.DMA] * _NBUF,
        compiler_params=_sc_params,
    )(_agg_kernel_body)


def _tc_scale1_body(x_ref, w_ref, degp_ref, y_ref, dis_ref):
    deg = degp_ref[0, : _N, :] + degp_ref[1, : _N, :] + 1.0
    dis = lax.rsqrt(deg)
    xw = jnp.dot(x_ref[...], w_ref[...], preferred_element_type=jnp.float32)
    y_ref[...] = xw * dis
    dis_ref[...] = dis


def _tc_mid_body(p_ref, y1_ref, dis_ref, b1_ref, y2_ref):
    dis = dis_ref[...]
    agg = p_ref[0, : _N, :] + p_ref[1, : _N, :] + y1_ref[...]
    h = jnp.maximum(agg * dis + b1_ref[...], 0.0)
    y2_ref[...] = h * dis


def _tc_final_body(p_ref, y2_ref, dis_ref, b2_ref, w2_ref, o_ref):
    agg = p_ref[0, : _N, :] + p_ref[1, : _N, :] + y2_ref[...]
    z = jnp.dot(agg * dis_ref[...], w2_ref[...],
                preferred_element_type=jnp.float32) + b2_ref[...]
    m = jnp.max(z, axis=1, keepdims=True)
    lse = jnp.log(jnp.sum(jnp.exp(z - m), axis=1, keepdims=True)) + m
    o_ref[...] = z - lse


def kernel(x, edge_index, W1, b1, W2, b2):
    n, d_in = x.shape
    d_hid = W1.shape[1]
    d_out = W2.shape[1]
    e = edge_index.shape[1]

    src = edge_index[0].astype(jnp.int32)
    dst = edge_index[1].astype(jnp.int32)
    k_slices = -(-e // (_NW * _SL))
    k_slices = -(-k_slices // _NBUF) * _NBUF
    e_pad = _NW * _SL * k_slices
    src_p = jnp.concatenate([src, jnp.zeros((e_pad - e,), jnp.int32)])
    dst_p = jnp.concatenate([dst, jnp.full((e_pad - e,), _N, jnp.int32)])
    src3 = src_p.reshape(_NW, k_slices, _SL)
    dst3 = dst_p.reshape(_NW, k_slices, _SL)
    zeros_hbm = jnp.zeros((_RPT, max(d_hid, d_out)), jnp.float32)

    degp = _make_deg(k_slices)(dst3, zeros_hbm[:, 0], jnp.ones((_SL,), jnp.float32))

    y1, dis = pl.pallas_call(
        _tc_scale1_body,
        out_shape=(
            jax.ShapeDtypeStruct((n, d_hid), jnp.float32),
            jax.ShapeDtypeStruct((n, 1), jnp.float32),
        ),
    )(x, W1, degp.reshape(_NC, _ACC_ROWS, 1))

    p1 = _make_agg(d_hid, k_slices)(y1, src3, dst3, zeros_hbm[:, :d_hid])

    y2 = pl.pallas_call(
        _tc_mid_body,
        out_shape=jax.ShapeDtypeStruct((n, d_hid), jnp.float32),
    )(p1, y1, dis, b1.reshape(1, d_hid))

    p2 = _make_agg(d_hid, k_slices)(y2, src3, dst3, zeros_hbm[:, :d_hid])

    out = pl.pallas_call(
        _tc_final_body,
        out_shape=jax.ShapeDtypeStruct((n, d_out), jnp.float32),
    )(p2, y2, dis, b2.reshape(1, d_out), W2)
    return out


# speedup vs baseline: 2.0559x; 1.4101x over previous
import functools

import jax
import jax.numpy as jnp
from jax import lax
from jax.experimental import pallas as pl
from jax.experimental.pallas import tpu as pltpu
from jax.experimental.pallas import tpu_sc as plsc

_N = 10000
_NC = 2
_NS = 16
_NW = _NC * _NS
_SL = 128
_ACC_ROWS = 10240
_RPT = _ACC_ROWS // _NS

_mesh = plsc.VectorSubcoreMesh(core_axis_name="c", subcore_axis_name="s")
_sc_params = pltpu.CompilerParams(use_tc_tiling_on_sc=False)


def _deg_kernel_body(didx_hbm, zeros_hbm, ones_hbm, out_hbm,
                     didx_v, ones_v, acc_sh, sem):
    cid = lax.axis_index("c")
    sid = lax.axis_index("s")
    wid = cid * _NS + sid
    k = didx_hbm.shape[1]

    pltpu.sync_copy(zeros_hbm.at[pl.ds(0, _RPT)], acc_sh.at[pl.ds(sid * _RPT, _RPT)])
    pltpu.sync_copy(ones_hbm, ones_v)
    pltpu.sync_copy(didx_hbm.at[wid], didx_v)
    plsc.subcore_barrier()

    def body(j, carry):
        pltpu.sync_copy(ones_v, acc_sh.at[didx_v.at[j]], add=True)
        return carry

    lax.fori_loop(0, k, body, 0, unroll=False)
    plsc.subcore_barrier()
    pltpu.sync_copy(acc_sh.at[pl.ds(sid * _RPT, _RPT)],
                    out_hbm.at[cid, pl.ds(sid * _RPT, _RPT)])


def _make_deg(k_slices):
    return functools.partial(
        pl.kernel,
        out_type=jax.ShapeDtypeStruct((_NC, _ACC_ROWS), jnp.float32),
        mesh=_mesh,
        scratch_types=[
            pltpu.VMEM((k_slices, _SL), jnp.int32),
            pltpu.VMEM((_SL,), jnp.float32),
            pltpu.VMEM_SHARED((_ACC_ROWS,), jnp.float32),
            pltpu.SemaphoreType.DMA,
        ],
        compiler_params=_sc_params,
    )(_deg_kernel_body)


_NBUF = 4


def _agg_kernel_body(y_hbm, sidx_hbm, didx_hbm, zeros_hbm, out_hbm,
                     sidx_v, didx_v, rows_v, y_sh, acc_sh, *sems):
    cid = lax.axis_index("c")
    sid = lax.axis_index("s")
    wid = cid * _NS + sid
    k = sidx_hbm.shape[1]
    groups = k // _NBUF
    n_rows = y_hbm.shape[0]
    rps = n_rows // _NS

    pltpu.sync_copy(zeros_hbm.at[pl.ds(0, _RPT)], acc_sh.at[pl.ds(sid * _RPT, _RPT)])
    pltpu.sync_copy(y_hbm.at[pl.ds(sid * rps, rps)], y_sh.at[pl.ds(sid * rps, rps)])
    pltpu.sync_copy(sidx_hbm.at[wid], sidx_v)
    pltpu.sync_copy(didx_hbm.at[wid], didx_v)
    plsc.subcore_barrier()

    for b in range(_NBUF):
        pltpu.async_copy(y_sh.at[sidx_v.at[b]], rows_v.at[b], sems[b])

    def body(g, carry):
        for b in range(_NBUF):
            s = g * _NBUF + b
            pltpu.make_async_copy(y_sh.at[sidx_v.at[s]], rows_v.at[b], sems[b]).wait()
            pltpu.async_copy(rows_v.at[b], acc_sh.at[didx_v.at[s]], sems[b], add=True).wait()
            pltpu.async_copy(y_sh.at[sidx_v.at[s + _NBUF]], rows_v.at[b], sems[b])
        return carry

    lax.fori_loop(0, groups - 1, body, 0, unroll=False)
    for b in range(_NBUF):
        s = (groups - 1) * _NBUF + b
        pltpu.make_async_copy(y_sh.at[sidx_v.at[s]], rows_v.at[b], sems[b]).wait()
        pltpu.async_copy(rows_v.at[b], acc_sh.at[didx_v.at[s]], sems[b], add=True).wait()

    plsc.subcore_barrier()
    pltpu.sync_copy(acc_sh.at[pl.ds(sid * _RPT, _RPT)],
                    out_hbm.at[cid, pl.ds(sid * _RPT, _RPT)])


def _make_agg(feat, k_slices, n_rows):
    return functools.partial(
        pl.kernel,
        out_type=jax.ShapeDtypeStruct((_NC, _ACC_ROWS, feat), jnp.float32),
        mesh=_mesh,
        scratch_types=[
            pltpu.VMEM((k_slices, _SL), jnp.int32),
            pltpu.VMEM((k_slices, _SL), jnp.int32),
            pltpu.VMEM((_NBUF, _SL, feat), jnp.float32),
            pltpu.VMEM_SHARED((n_rows, feat), jnp.float32),
            pltpu.VMEM_SHARED((_ACC_ROWS, feat), jnp.float32),
        ] + [pltpu.SemaphoreType.DMA] * _NBUF,
        compiler_params=_sc_params,
    )(_agg_kernel_body)


def _tc_scale1_body(x_ref, w_ref, degp_ref, y_ref, dis_ref):
    deg = degp_ref[0, : _N, :] + degp_ref[1, : _N, :] + 1.0
    dis = lax.rsqrt(deg)
    xw = jnp.dot(x_ref[...], w_ref[...], preferred_element_type=jnp.float32)
    y_ref[...] = xw * dis
    dis_ref[...] = dis


def _tc_mid_body(p_ref, y1_ref, dis_ref, b1_ref, y2_ref):
    dis = dis_ref[...]
    agg = p_ref[0, : _N, :] + p_ref[1, : _N, :] + y1_ref[...]
    h = jnp.maximum(agg * dis + b1_ref[...], 0.0)
    y2_ref[...] = h * dis


def _tc_final_body(p_ref, y2_ref, dis_ref, b2_ref, w2_ref, o_ref):
    agg = p_ref[0, : _N, :] + p_ref[1, : _N, :] + y2_ref[...]
    z = jnp.dot(agg * dis_ref[...], w2_ref[...],
                preferred_element_type=jnp.float32) + b2_ref[...]
    m = jnp.max(z, axis=1, keepdims=True)
    lse = jnp.log(jnp.sum(jnp.exp(z - m), axis=1, keepdims=True)) + m
    o_ref[...] = z - lse


def kernel(x, edge_index, W1, b1, W2, b2):
    n, d_in = x.shape
    d_hid = W1.shape[1]
    d_out = W2.shape[1]
    e = edge_index.shape[1]

    src = edge_index[0].astype(jnp.int32)
    dst = edge_index[1].astype(jnp.int32)
    k_slices = -(-e // (_NW * _SL))
    k_slices = -(-k_slices // _NBUF) * _NBUF
    e_pad = _NW * _SL * k_slices
    src_p = jnp.concatenate([src, jnp.zeros((e_pad - e,), jnp.int32)])
    dst_p = jnp.concatenate([dst, jnp.full((e_pad - e,), _N, jnp.int32)])
    src3 = src_p.reshape(_NW, k_slices, _SL)
    dst3 = dst_p.reshape(_NW, k_slices, _SL)
    zeros_hbm = jnp.zeros((_RPT, max(d_hid, d_out)), jnp.float32)

    degp = _make_deg(k_slices)(dst3, zeros_hbm[:, 0], jnp.ones((_SL,), jnp.float32))

    y1, dis = pl.pallas_call(
        _tc_scale1_body,
        out_shape=(
            jax.ShapeDtypeStruct((n, d_hid), jnp.float32),
            jax.ShapeDtypeStruct((n, 1), jnp.float32),
        ),
    )(x, W1, degp.reshape(_NC, _ACC_ROWS, 1))

    p1 = _make_agg(d_hid, k_slices, n)(y1, src3, dst3, zeros_hbm[:, :d_hid])

    y2 = pl.pallas_call(
        _tc_mid_body,
        out_shape=jax.ShapeDtypeStruct((n, d_hid), jnp.float32),
    )(p1, y1, dis, b1.reshape(1, d_hid))

    p2 = _make_agg(d_hid, k_slices, n)(y2, src3, dst3, zeros_hbm[:, :d_hid])

    out = pl.pallas_call(
        _tc_final_body,
        out_shape=jax.ShapeDtypeStruct((n, d_out), jnp.float32),
    )(p2, y2, dis, b2.reshape(1, d_out), W2)
    return out


# speedup vs baseline: 2.2360x; 1.0876x over previous
import functools

import jax
import jax.numpy as jnp
from jax import lax
from jax.experimental import pallas as pl
from jax.experimental.pallas import tpu as pltpu
from jax.experimental.pallas import tpu_sc as plsc

_N = 10000
_NC = 2
_NS = 16
_NW = _NC * _NS
_SL = 125
_ACC_ROWS = 10240
_RPT = _ACC_ROWS // _NS
_DW = 8
_NBUF = 4

_mesh = plsc.VectorSubcoreMesh(core_axis_name="c", subcore_axis_name="s")
_sc_params = pltpu.CompilerParams(use_tc_tiling_on_sc=False)


def _deg_kernel_body(ei_hbm, zeros_hbm, ones_hbm, out_hbm,
                     didx_v, ones_v, acc_sh, sem):
    cid = lax.axis_index("c")
    sid = lax.axis_index("s")
    wid = cid * _NS + sid
    k = didx_v.shape[0]

    pltpu.sync_copy(zeros_hbm, acc_sh.at[pl.ds(sid * _RPT, _RPT)])
    pltpu.sync_copy(ones_hbm, ones_v)
    pltpu.sync_copy(ei_hbm.at[1].at[pl.ds(wid * k, k)], didx_v)
    plsc.subcore_barrier()

    def body(j, carry):
        pltpu.sync_copy(ones_v, acc_sh.at[didx_v.at[j]], add=True)
        return carry

    lax.fori_loop(0, k, body, 0, unroll=False)
    plsc.subcore_barrier()
    pltpu.sync_copy(acc_sh.at[pl.ds(sid * _RPT, _RPT)],
                    out_hbm.at[cid, pl.ds(sid * _RPT, _RPT)])


def _make_deg(k_slices):
    return functools.partial(
        pl.kernel,
        out_type=jax.ShapeDtypeStruct((_NC, _ACC_ROWS, _DW), jnp.float32),
        mesh=_mesh,
        scratch_types=[
            pltpu.VMEM((k_slices, _SL), jnp.int32),
            pltpu.VMEM((_SL, _DW), jnp.float32),
            pltpu.VMEM_SHARED((_ACC_ROWS, _DW), jnp.float32),
            pltpu.SemaphoreType.DMA,
        ],
        compiler_params=_sc_params,
    )(_deg_kernel_body)


def _agg_kernel_body(y_hbm, ei_hbm, zeros_hbm, out_hbm,
                     sidx_v, didx_v, rows_v, y_sh, acc_sh, *sems):
    cid = lax.axis_index("c")
    sid = lax.axis_index("s")
    wid = cid * _NS + sid
    k = sidx_v.shape[0]
    groups = k // _NBUF
    n_rows = y_hbm.shape[0]
    rps = n_rows // _NS

    pltpu.sync_copy(zeros_hbm, acc_sh.at[pl.ds(sid * _RPT, _RPT)])
    pltpu.sync_copy(y_hbm.at[pl.ds(sid * rps, rps)], y_sh.at[pl.ds(sid * rps, rps)])
    pltpu.sync_copy(ei_hbm.at[0].at[pl.ds(wid * k, k)], sidx_v)
    pltpu.sync_copy(ei_hbm.at[1].at[pl.ds(wid * k, k)], didx_v)
    plsc.subcore_barrier()

    for b in range(_NBUF):
        pltpu.async_copy(y_sh.at[sidx_v.at[b]], rows_v.at[b], sems[b])

    def body(g, carry):
        for b in range(_NBUF):
            s = g * _NBUF + b
            pltpu.make_async_copy(y_sh.at[sidx_v.at[s]], rows_v.at[b], sems[b]).wait()
            pltpu.async_copy(rows_v.at[b], acc_sh.at[didx_v.at[s]], sems[b], add=True).wait()
            pltpu.async_copy(y_sh.at[sidx_v.at[s + _NBUF]], rows_v.at[b], sems[b])
        return carry

    lax.fori_loop(0, groups - 1, body, 0, unroll=False)
    for b in range(_NBUF):
        s = (groups - 1) * _NBUF + b
        pltpu.make_async_copy(y_sh.at[sidx_v.at[s]], rows_v.at[b], sems[b]).wait()
        pltpu.async_copy(rows_v.at[b], acc_sh.at[didx_v.at[s]], sems[b], add=True).wait()

    plsc.subcore_barrier()
    pltpu.sync_copy(acc_sh.at[pl.ds(sid * _RPT, _RPT)],
                    out_hbm.at[cid, pl.ds(sid * _RPT, _RPT)])


def _make_agg(feat, k_slices, n_rows):
    return functools.partial(
        pl.kernel,
        out_type=jax.ShapeDtypeStruct((_NC, _ACC_ROWS, feat), jnp.float32),
        mesh=_mesh,
        scratch_types=[
            pltpu.VMEM((k_slices, _SL), jnp.int32),
            pltpu.VMEM((k_slices, _SL), jnp.int32),
            pltpu.VMEM((_NBUF, _SL, feat), jnp.float32),
            pltpu.VMEM_SHARED((n_rows, feat), jnp.float32),
            pltpu.VMEM_SHARED((_ACC_ROWS, feat), jnp.float32),
        ] + [pltpu.SemaphoreType.DMA] * _NBUF,
        compiler_params=_sc_params,
    )(_agg_kernel_body)


def _tc_scale1_body(x_ref, w_ref, degp_ref, y_ref, dis_ref):
    deg = degp_ref[0, : _N, 0:1] + degp_ref[1, : _N, 0:1] + 1.0
    dis = lax.rsqrt(deg)
    xw = jnp.dot(x_ref[...], w_ref[...], preferred_element_type=jnp.float32)
    y_ref[...] = xw * dis
    dis_ref[...] = dis


def _tc_mid_body(p_ref, y1_ref, dis_ref, b1_ref, y2_ref):
    dis = dis_ref[...]
    agg = p_ref[0, : _N, :] + p_ref[1, : _N, :] + y1_ref[...]
    h = jnp.maximum(agg * dis + b1_ref[...], 0.0)
    y2_ref[...] = h * dis


def _tc_final_body(p_ref, y2_ref, dis_ref, b2_ref, w2_ref, o_ref):
    agg = p_ref[0, : _N, :] + p_ref[1, : _N, :] + y2_ref[...]
    z = jnp.dot(agg * dis_ref[...], w2_ref[...],
                preferred_element_type=jnp.float32) + b2_ref[...]
    m = jnp.max(z, axis=1, keepdims=True)
    lse = jnp.log(jnp.sum(jnp.exp(z - m), axis=1, keepdims=True)) + m
    o_ref[...] = z - lse


def kernel(x, edge_index, W1, b1, W2, b2):
    n, d_in = x.shape
    d_hid = W1.shape[1]
    d_out = W2.shape[1]
    e = edge_index.shape[1]

    k_slices = e // (_NW * _SL)
    ei3 = edge_index.astype(jnp.int32).reshape(2, _NW * k_slices, _SL)
    zeros_d = jnp.zeros((_RPT, _DW), jnp.float32)
    zeros_f = jnp.zeros((_RPT, d_hid), jnp.float32)
    ones_d = jnp.ones((_SL, _DW), jnp.float32)

    degp = _make_deg(k_slices)(ei3, zeros_d, ones_d)

    y1, dis = pl.pallas_call(
        _tc_scale1_body,
        out_shape=(
            jax.ShapeDtypeStruct((n, d_hid), jnp.float32),
            jax.ShapeDtypeStruct((n, 1), jnp.float32),
        ),
    )(x, W1, degp)

    p1 = _make_agg(d_hid, k_slices, n)(y1, ei3, zeros_f)

    y2 = pl.pallas_call(
        _tc_mid_body,
        out_shape=jax.ShapeDtypeStruct((n, d_hid), jnp.float32),
    )(p1, y1, dis, b1.reshape(1, d_hid))

    p2 = _make_agg(d_hid, k_slices, n)(y2, ei3, zeros_f)

    out = pl.pallas_call(
        _tc_final_body,
        out_shape=jax.ShapeDtypeStruct((n, d_out), jnp.float32),
    )(p2, y2, dis, b2.reshape(1, d_out), W2)
    return out


# speedup vs baseline: 2.7908x; 1.2481x over previous
import functools

import jax
import jax.numpy as jnp
from jax import lax
from jax.experimental import pallas as pl
from jax.experimental.pallas import tpu as pltpu
from jax.experimental.pallas import tpu_sc as plsc

_N = 10000
_NC = 2
_NS = 16
_NW = _NC * _NS
_SL = 125
_ACC_ROWS = 10240
_RPT = _ACC_ROWS // _NS
_DW = 16
_NBUF = 4

_mesh = plsc.VectorSubcoreMesh(core_axis_name="c", subcore_axis_name="s")
_sc_params = pltpu.CompilerParams(use_tc_tiling_on_sc=False)


def _deg_kernel_body(ei_hbm, zeros_hbm, ones_hbm, out_hbm,
                     didx_v, ones_v, acc_sh, sem):
    cid = lax.axis_index("c")
    sid = lax.axis_index("s")
    wid = cid * _NS + sid
    k = didx_v.shape[0]

    pltpu.sync_copy(zeros_hbm, acc_sh.at[pl.ds(sid * _RPT, _RPT)])
    pltpu.sync_copy(ones_hbm, ones_v)
    pltpu.sync_copy(ei_hbm.at[1].at[pl.ds(wid * k, k)], didx_v)
    plsc.subcore_barrier()

    def body(j, carry):
        pltpu.sync_copy(ones_v, acc_sh.at[didx_v.at[j]], add=True)
        return carry

    lax.fori_loop(0, k, body, 0, unroll=False)
    plsc.subcore_barrier()
    pltpu.sync_copy(acc_sh.at[pl.ds(sid * _RPT, _RPT)],
                    out_hbm.at[cid, pl.ds(sid * _RPT, _RPT)])


def _make_deg(k_slices):
    return functools.partial(
        pl.kernel,
        out_type=jax.ShapeDtypeStruct((_NC, _ACC_ROWS, _DW), jnp.float32),
        mesh=_mesh,
        scratch_types=[
            pltpu.VMEM((k_slices, _SL), jnp.int32),
            pltpu.VMEM((_SL, _DW), jnp.float32),
            pltpu.VMEM_SHARED((_ACC_ROWS, _DW), jnp.float32),
            pltpu.SemaphoreType.DMA,
        ],
        compiler_params=_sc_params,
    )(_deg_kernel_body)


def _agg_kernel_body(y_hbm, ei_hbm, zeros_hbm, out_hbm,
                     sidx_v, didx_v, rows_v, y_sh, acc_sh, *sems):
    cid = lax.axis_index("c")
    sid = lax.axis_index("s")
    wid = cid * _NS + sid
    k = sidx_v.shape[0]
    groups = k // _NBUF
    n_rows = y_hbm.shape[0]
    rps = n_rows // _NS

    pltpu.sync_copy(zeros_hbm, acc_sh.at[pl.ds(sid * _RPT, _RPT)])
    pltpu.sync_copy(y_hbm.at[pl.ds(sid * rps, rps)], y_sh.at[pl.ds(sid * rps, rps)])
    pltpu.sync_copy(ei_hbm.at[0].at[pl.ds(wid * k, k)], sidx_v)
    pltpu.sync_copy(ei_hbm.at[1].at[pl.ds(wid * k, k)], didx_v)
    plsc.subcore_barrier()

    for b in range(_NBUF):
        pltpu.async_copy(y_sh.at[sidx_v.at[b]], rows_v.at[b], sems[b])

    def body(g, carry):
        for b in range(_NBUF):
            s = g * _NBUF + b
            pltpu.make_async_copy(y_sh.at[sidx_v.at[s]], rows_v.at[b], sems[b]).wait()
            pltpu.async_copy(rows_v.at[b], acc_sh.at[didx_v.at[s]], sems[b], add=True).wait()
            pltpu.async_copy(y_sh.at[sidx_v.at[s + _NBUF]], rows_v.at[b], sems[b])
        return carry

    lax.fori_loop(0, groups - 1, body, 0, unroll=False)
    for b in range(_NBUF):
        s = (groups - 1) * _NBUF + b
        pltpu.make_async_copy(y_sh.at[sidx_v.at[s]], rows_v.at[b], sems[b]).wait()
        pltpu.async_copy(rows_v.at[b], acc_sh.at[didx_v.at[s]], sems[b], add=True).wait()

    plsc.subcore_barrier()
    pltpu.sync_copy(acc_sh.at[pl.ds(sid * _RPT, _RPT)],
                    out_hbm.at[cid, pl.ds(sid * _RPT, _RPT)])


def _make_agg(feat, k_slices, n_rows):
    return functools.partial(
        pl.kernel,
        out_type=jax.ShapeDtypeStruct((_NC, _ACC_ROWS, feat), jnp.float32),
        mesh=_mesh,
        scratch_types=[
            pltpu.VMEM((k_slices, _SL), jnp.int32),
            pltpu.VMEM((k_slices, _SL), jnp.int32),
            pltpu.VMEM((_NBUF, _SL, feat), jnp.float32),
            pltpu.VMEM_SHARED((n_rows, feat), jnp.float32),
            pltpu.VMEM_SHARED((_ACC_ROWS, feat), jnp.float32),
        ] + [pltpu.SemaphoreType.DMA] * _NBUF,
        compiler_params=_sc_params,
    )(_agg_kernel_body)


_GRP = 16
_NV = _N // _GRP


def _tc_scale1_body(x_ref, w1b_ref, degp_ref, y_ref, disb_ref):
    disb = lax.rsqrt(degp_ref[0] + degp_ref[1] + 1.0)
    disb_ref[...] = disb
    xw = jnp.dot(x_ref[...], w1b_ref[...], preferred_element_type=jnp.float32)
    y_ref[...] = xw * disb[: _NV]


def _tc_mid_body(p_ref, y1_ref, disb_ref, b1_ref, y2_ref):
    dis = disb_ref[: _NV, :]
    agg = p_ref[0, : _NV, :] + p_ref[1, : _NV, :] + y1_ref[...]
    h = jnp.maximum(agg * dis + b1_ref[...], 0.0)
    y2_ref[...] = h * dis


def _tc_final_body(p_ref, y2_ref, disb_ref, b2_ref, w2b_ref, s_ref, b_ref,
                   o_ref):
    agg = p_ref[0, : _NV, :] + p_ref[1, : _NV, :] + y2_ref[...]
    z = jnp.dot(agg * disb_ref[: _NV, :], w2b_ref[...],
                preferred_element_type=jnp.float32) + b2_ref[...]
    c = jnp.max(z, axis=1, keepdims=True)
    ez = jnp.exp(z - c)
    s = jnp.dot(ez, s_ref[...], preferred_element_type=jnp.float32)
    lse = jnp.dot(jnp.log(s), b_ref[...], preferred_element_type=jnp.float32)
    o_ref[...] = (z - c) - lse


def kernel(x, edge_index, W1, b1, W2, b2):
    n, d_in = x.shape
    d_hid = W1.shape[1]
    d_out = W2.shape[1]
    e = edge_index.shape[1]

    k_slices = e // (_NW * _SL)
    ei3 = edge_index.astype(jnp.int32).reshape(2, _NW * k_slices, _SL)
    zeros_d = jnp.zeros((_RPT, _DW), jnp.float32)
    zeros_f = jnp.zeros((_RPT, d_hid), jnp.float32)
    ones_d = jnp.ones((_SL, _DW), jnp.float32)

    nv = n // _GRP
    accv = _ACC_ROWS // _GRP
    vw = _GRP * d_hid
    ow = _GRP * d_out
    eye = jnp.eye(_GRP, dtype=jnp.float32)
    w1b = jnp.kron(eye, W1)
    w2b = jnp.kron(eye, W2)
    seg_s = jnp.kron(eye, jnp.ones((d_out, 1), jnp.float32))
    seg_b = jnp.kron(eye, jnp.ones((1, d_out), jnp.float32))
    b1t = jnp.tile(b1, _GRP).reshape(1, vw)
    b2t = jnp.tile(b2, _GRP).reshape(1, ow)

    degp = _make_deg(k_slices)(ei3, zeros_d, ones_d)

    y1v, disb = pl.pallas_call(
        _tc_scale1_body,
        out_shape=(
            jax.ShapeDtypeStruct((nv, vw), jnp.float32),
            jax.ShapeDtypeStruct((accv, vw), jnp.float32),
        ),
    )(x.reshape(nv, _GRP * d_in), w1b, degp.reshape(_NC, accv, vw))

    p1 = _make_agg(d_hid, k_slices, n)(y1v.reshape(n, d_hid), ei3, zeros_f)

    y2v = pl.pallas_call(
        _tc_mid_body,
        out_shape=jax.ShapeDtypeStruct((nv, vw), jnp.float32),
    )(p1.reshape(_NC, accv, vw), y1v, disb, b1t)

    p2 = _make_agg(d_hid, k_slices, n)(y2v.reshape(n, d_hid), ei3, zeros_f)

    outv = pl.pallas_call(
        _tc_final_body,
        out_shape=jax.ShapeDtypeStruct((nv, ow), jnp.float32),
    )(p2.reshape(_NC, accv, vw), y2v, disb, b2t, w2b, seg_s, seg_b)
    return outv.reshape(n, d_out)


# speedup vs baseline: 3.0474x; 1.0920x over previous
import functools

import jax
import jax.numpy as jnp
from jax import lax
from jax.experimental import pallas as pl
from jax.experimental.pallas import tpu as pltpu
from jax.experimental.pallas import tpu_sc as plsc

_N = 10000
_NC = 2
_NS = 16
_NW = _NC * _NS
_SL = 500
_ACC_ROWS = 10240
_RPT = _ACC_ROWS // _NS
_DW = 16
_NBUF = 2

_mesh = plsc.VectorSubcoreMesh(core_axis_name="c", subcore_axis_name="s")
_sc_params = pltpu.CompilerParams(use_tc_tiling_on_sc=False)


def _deg_kernel_body(ei_hbm, zeros_hbm, ones_hbm, out_hbm,
                     didx_v, ones_v, acc_sh, sem):
    cid = lax.axis_index("c")
    sid = lax.axis_index("s")
    wid = cid * _NS + sid
    k = didx_v.shape[0]

    pltpu.sync_copy(zeros_hbm, acc_sh.at[pl.ds(sid * _RPT, _RPT)])
    pltpu.sync_copy(ones_hbm, ones_v)
    pltpu.sync_copy(ei_hbm.at[1].at[pl.ds(wid * k, k)], didx_v)
    plsc.subcore_barrier()

    def body(j, carry):
        pltpu.sync_copy(ones_v, acc_sh.at[didx_v.at[j]], add=True)
        return carry

    lax.fori_loop(0, k, body, 0, unroll=False)
    plsc.subcore_barrier()
    pltpu.sync_copy(acc_sh.at[pl.ds(sid * _RPT, _RPT)],
                    out_hbm.at[cid, pl.ds(sid * _RPT, _RPT)])


def _make_deg(k_slices):
    return functools.partial(
        pl.kernel,
        out_type=jax.ShapeDtypeStruct((_NC, _ACC_ROWS, _DW), jnp.float32),
        mesh=_mesh,
        scratch_types=[
            pltpu.VMEM((k_slices, _SL), jnp.int32),
            pltpu.VMEM((_SL, _DW), jnp.float32),
            pltpu.VMEM_SHARED((_ACC_ROWS, _DW), jnp.float32),
            pltpu.SemaphoreType.DMA,
        ],
        compiler_params=_sc_params,
    )(_deg_kernel_body)


def _agg_kernel_body(y_hbm, ei_hbm, zeros_hbm, out_hbm,
                     sidx_v, didx_v, rows_v, y_sh, acc_sh, *sems):
    cid = lax.axis_index("c")
    sid = lax.axis_index("s")
    wid = cid * _NS + sid
    k = sidx_v.shape[0]
    groups = k // _NBUF
    n_rows = y_hbm.shape[0]
    rps = n_rows // _NS

    pltpu.sync_copy(zeros_hbm, acc_sh.at[pl.ds(sid * _RPT, _RPT)])
    pltpu.sync_copy(y_hbm.at[pl.ds(sid * rps, rps)], y_sh.at[pl.ds(sid * rps, rps)])
    pltpu.sync_copy(ei_hbm.at[0].at[pl.ds(wid * k, k)], sidx_v)
    pltpu.sync_copy(ei_hbm.at[1].at[pl.ds(wid * k, k)], didx_v)
    plsc.subcore_barrier()

    for b in range(_NBUF):
        pltpu.async_copy(y_sh.at[sidx_v.at[b]], rows_v.at[b], sems[b])

    def body(g, carry):
        for b in range(_NBUF):
            s = g * _NBUF + b
            pltpu.make_async_copy(y_sh.at[sidx_v.at[s]], rows_v.at[b], sems[b]).wait()
            pltpu.async_copy(rows_v.at[b], acc_sh.at[didx_v.at[s]], sems[b], add=True).wait()
            pltpu.async_copy(y_sh.at[sidx_v.at[s + _NBUF]], rows_v.at[b], sems[b])
        return carry

    lax.fori_loop(0, groups - 1, body, 0, unroll=False)
    for b in range(_NBUF):
        s = (groups - 1) * _NBUF + b
        pltpu.make_async_copy(y_sh.at[sidx_v.at[s]], rows_v.at[b], sems[b]).wait()
        pltpu.async_copy(rows_v.at[b], acc_sh.at[didx_v.at[s]], sems[b], add=True).wait()

    plsc.subcore_barrier()
    pltpu.sync_copy(acc_sh.at[pl.ds(sid * _RPT, _RPT)],
                    out_hbm.at[cid, pl.ds(sid * _RPT, _RPT)])


def _make_agg(feat, k_slices, n_rows):
    return functools.partial(
        pl.kernel,
        out_type=jax.ShapeDtypeStruct((_NC, _ACC_ROWS, feat), jnp.float32),
        mesh=_mesh,
        scratch_types=[
            pltpu.VMEM((k_slices, _SL), jnp.int32),
            pltpu.VMEM((k_slices, _SL), jnp.int32),
            pltpu.VMEM((_NBUF, _SL, feat), jnp.float32),
            pltpu.VMEM_SHARED((n_rows, feat), jnp.float32),
            pltpu.VMEM_SHARED((_ACC_ROWS, feat), jnp.float32),
        ] + [pltpu.SemaphoreType.DMA] * _NBUF,
        compiler_params=_sc_params,
    )(_agg_kernel_body)


_GRP = 8
_NV = _N // _GRP


def _tc_scale1_body(x_ref, w1b_ref, degp_ref, y_ref, disb_ref):
    disb = lax.rsqrt(degp_ref[0] + degp_ref[1] + 1.0)
    disb_ref[...] = disb
    xw = jnp.dot(x_ref[...], w1b_ref[...], preferred_element_type=jnp.float32)
    y_ref[...] = xw * disb[: _NV]


def _tc_mid_body(p_ref, y1_ref, disb_ref, b1_ref, y2_ref):
    dis = disb_ref[: _NV, :]
    agg = p_ref[0, : _NV, :] + p_ref[1, : _NV, :] + y1_ref[...]
    h = jnp.maximum(agg * dis + b1_ref[...], 0.0)
    y2_ref[...] = h * dis


def _tc_final_body(p_ref, y2_ref, disb_ref, b2_ref, w2b_ref, s_ref, b_ref,
                   o_ref):
    agg = p_ref[0, : _NV, :] + p_ref[1, : _NV, :] + y2_ref[...]
    z = jnp.dot(agg * disb_ref[: _NV, :], w2b_ref[...],
                preferred_element_type=jnp.float32) + b2_ref[...]
    c = jnp.max(z, axis=1, keepdims=True)
    ez = jnp.exp(z - c)
    s = jnp.dot(ez, s_ref[...], preferred_element_type=jnp.float32,
                precision=lax.Precision.HIGHEST)
    lse = jnp.dot(jnp.log(s), b_ref[...], preferred_element_type=jnp.float32,
                  precision=lax.Precision.HIGHEST)
    o_ref[...] = (z - c) - lse


def kernel(x, edge_index, W1, b1, W2, b2):
    n, d_in = x.shape
    d_hid = W1.shape[1]
    d_out = W2.shape[1]
    e = edge_index.shape[1]

    k_slices = e // (_NW * _SL)
    ei3 = edge_index.astype(jnp.int32).reshape(2, _NW * k_slices, _SL)
    zeros_d = jnp.zeros((_RPT, _DW), jnp.float32)
    zeros_f = jnp.zeros((_RPT, d_hid), jnp.float32)
    ones_d = jnp.ones((_SL, _DW), jnp.float32)

    nv = n // _GRP
    accv = _ACC_ROWS // _GRP
    vw = _GRP * d_hid
    ow = _GRP * d_out
    eye = jnp.eye(_GRP, dtype=jnp.float32)
    w1b = jnp.kron(eye, W1)
    w2b = jnp.kron(eye, W2)
    seg_s = jnp.kron(eye, jnp.ones((d_out, 1), jnp.float32))
    seg_b = jnp.kron(eye, jnp.ones((1, d_out), jnp.float32))
    b1t = jnp.tile(b1, _GRP).reshape(1, vw)
    b2t = jnp.tile(b2, _GRP).reshape(1, ow)

    degp = _make_deg(k_slices)(ei3, zeros_d, ones_d)

    y1v, disb = pl.pallas_call(
        _tc_scale1_body,
        out_shape=(
            jax.ShapeDtypeStruct((nv, vw), jnp.float32),
            jax.ShapeDtypeStruct((accv, vw), jnp.float32),
        ),
    )(x.reshape(nv, _GRP * d_in), w1b, degp.reshape(_NC, accv, vw))

    p1 = _make_agg(d_hid, k_slices, n)(y1v.reshape(n, d_hid), ei3, zeros_f)

    y2v = pl.pallas_call(
        _tc_mid_body,
        out_shape=jax.ShapeDtypeStruct((nv, vw), jnp.float32),
    )(p1.reshape(_NC, accv, vw), y1v, disb, b1t)

    p2 = _make_agg(d_hid, k_slices, n)(y2v.reshape(n, d_hid), ei3, zeros_f)

    outv = pl.pallas_call(
        _tc_final_body,
        out_shape=jax.ShapeDtypeStruct((nv, ow), jnp.float32),
    )(p2.reshape(_NC, accv, vw), y2v, disb, b2t, w2b, seg_s, seg_b)
    return outv.reshape(n, d_out)


# speedup vs baseline: 3.2334x; 1.0610x over previous
import functools

import jax
import jax.numpy as jnp
from jax import lax
from jax.experimental import pallas as pl
from jax.experimental.pallas import tpu as pltpu
from jax.experimental.pallas import tpu_sc as plsc

_N = 10000
_NC = 2
_NS = 16
_NW = _NC * _NS
_SL = 500
_ACC_ROWS = 10240
_RPT = _ACC_ROWS // _NS
_DW = 16
_NBUF = 2

_mesh = plsc.VectorSubcoreMesh(core_axis_name="c", subcore_axis_name="s")
_sc_params = pltpu.CompilerParams(use_tc_tiling_on_sc=False)


def _deg_kernel_body(ei_hbm, zeros_hbm, ones_hbm, out_hbm,
                     didx_v, ones_v, acc_sh, sem):
    cid = lax.axis_index("c")
    sid = lax.axis_index("s")
    wid = cid * _NS + sid
    k = didx_v.shape[0]

    pltpu.sync_copy(zeros_hbm, acc_sh.at[pl.ds(sid * _RPT, _RPT)])
    pltpu.sync_copy(ones_hbm, ones_v)
    pltpu.sync_copy(ei_hbm.at[1].at[pl.ds(wid * k, k)], didx_v)
    plsc.subcore_barrier()

    def body(j, carry):
        pltpu.sync_copy(ones_v, acc_sh.at[didx_v.at[j]], add=True)
        return carry

    lax.fori_loop(0, k, body, 0, unroll=False)
    plsc.subcore_barrier()
    pltpu.sync_copy(acc_sh.at[pl.ds(sid * _RPT, _RPT)],
                    out_hbm.at[cid, pl.ds(sid * _RPT, _RPT)])


def _make_deg(k_slices):
    return functools.partial(
        pl.kernel,
        out_type=jax.ShapeDtypeStruct((_NC, _ACC_ROWS, _DW), jnp.float32),
        mesh=_mesh,
        scratch_types=[
            pltpu.VMEM((k_slices, _SL), jnp.int32),
            pltpu.VMEM((_SL, _DW), jnp.float32),
            pltpu.VMEM_SHARED((_ACC_ROWS, _DW), jnp.float32),
            pltpu.SemaphoreType.DMA,
        ],
        compiler_params=_sc_params,
    )(_deg_kernel_body)


def _agg_kernel_body(y_hbm, ei_hbm, zeros_hbm, out_hbm,
                     sidx_v, didx_v, rows_v, y_sh, acc_sh, *sems):
    cid = lax.axis_index("c")
    sid = lax.axis_index("s")
    wid = cid * _NS + sid
    k = sidx_v.shape[0]
    groups = k // _NBUF
    n_rows = y_hbm.shape[0]
    rps = n_rows // _NS

    pltpu.sync_copy(zeros_hbm, acc_sh.at[pl.ds(sid * _RPT, _RPT)])
    pltpu.sync_copy(y_hbm.at[pl.ds(sid * rps, rps)], y_sh.at[pl.ds(sid * rps, rps)])
    pltpu.sync_copy(ei_hbm.at[0].at[pl.ds(wid * k, k)], sidx_v)
    pltpu.sync_copy(ei_hbm.at[1].at[pl.ds(wid * k, k)], didx_v)
    plsc.subcore_barrier()

    for b in range(_NBUF):
        pltpu.async_copy(y_sh.at[sidx_v.at[b]], rows_v.at[b], sems[b])

    def body(g, carry):
        for b in range(_NBUF):
            s = g * _NBUF + b
            pltpu.make_async_copy(y_sh.at[sidx_v.at[s]], rows_v.at[b], sems[b]).wait()
            pltpu.async_copy(rows_v.at[b], acc_sh.at[didx_v.at[s]], sems[b], add=True).wait()
            pltpu.async_copy(y_sh.at[sidx_v.at[s + _NBUF]], rows_v.at[b], sems[b])
        return carry

    lax.fori_loop(0, groups - 1, body, 0, unroll=False)
    for b in range(_NBUF):
        s = (groups - 1) * _NBUF + b
        pltpu.make_async_copy(y_sh.at[sidx_v.at[s]], rows_v.at[b], sems[b]).wait()
        pltpu.async_copy(rows_v.at[b], acc_sh.at[didx_v.at[s]], sems[b], add=True).wait()

    plsc.subcore_barrier()
    pltpu.sync_copy(acc_sh.at[pl.ds(sid * _RPT, _RPT)],
                    out_hbm.at[cid, pl.ds(sid * _RPT, _RPT)])


def _make_agg(feat, k_slices, n_rows):
    return functools.partial(
        pl.kernel,
        out_type=jax.ShapeDtypeStruct((_NC, _ACC_ROWS, feat), jnp.float32),
        mesh=_mesh,
        scratch_types=[
            pltpu.VMEM((k_slices, _SL), jnp.int32),
            pltpu.VMEM((k_slices, _SL), jnp.int32),
            pltpu.VMEM((_NBUF, _SL, feat), jnp.float32),
            pltpu.VMEM_SHARED((n_rows, feat), jnp.float32),
            pltpu.VMEM_SHARED((_ACC_ROWS, feat), jnp.float32),
        ] + [pltpu.SemaphoreType.DMA] * _NBUF,
        compiler_params=_sc_params,
    )(_agg_kernel_body)


_GRP = 8
_NV = _N // _GRP


def _tc_xw_body(x_ref, w1b_ref, xw_ref):
    xw_ref[...] = jnp.dot(x_ref[...], w1b_ref[...],
                          preferred_element_type=jnp.float32)


def _tc_scale1_body(xw_ref, degp_ref, y_ref, disb_ref):
    disb = lax.rsqrt(degp_ref[0] + degp_ref[1] + 1.0)
    disb_ref[...] = disb
    y_ref[...] = xw_ref[...] * disb[: _NV]


def _tc_mid_body(p_ref, y1_ref, disb_ref, b1_ref, y2_ref):
    dis = disb_ref[: _NV, :]
    agg = p_ref[0, : _NV, :] + p_ref[1, : _NV, :] + y1_ref[...]
    h = jnp.maximum(agg * dis + b1_ref[...], 0.0)
    y2_ref[...] = h * dis


def _tc_final_body(p_ref, y2_ref, disb_ref, b2_ref, w2b_ref, s_ref, b_ref,
                   o_ref):
    agg = p_ref[0, : _NV, :] + p_ref[1, : _NV, :] + y2_ref[...]
    z = jnp.dot(agg * disb_ref[: _NV, :], w2b_ref[...],
                preferred_element_type=jnp.float32) + b2_ref[...]
    c = jnp.max(z, axis=1, keepdims=True)
    ez = jnp.exp(z - c)
    s = jnp.dot(ez, s_ref[...], preferred_element_type=jnp.float32)
    lse = jnp.dot(jnp.log(s), b_ref[...], preferred_element_type=jnp.float32)
    o_ref[...] = (z - c) - lse


def kernel(x, edge_index, W1, b1, W2, b2):
    n, d_in = x.shape
    d_hid = W1.shape[1]
    d_out = W2.shape[1]
    e = edge_index.shape[1]

    k_slices = e // (_NW * _SL)
    ei3 = edge_index.astype(jnp.int32).reshape(2, _NW * k_slices, _SL)
    zeros_d = jnp.zeros((_RPT, _DW), jnp.float32)
    zeros_f = jnp.zeros((_RPT, d_hid), jnp.float32)
    ones_d = jnp.ones((_SL, _DW), jnp.float32)

    nv = n // _GRP
    accv = _ACC_ROWS // _GRP
    vw = _GRP * d_hid
    ow = _GRP * d_out
    eye = jnp.eye(_GRP, dtype=jnp.float32)
    w1b = jnp.kron(eye, W1)
    w2b = jnp.kron(eye, W2)
    seg_s = jnp.kron(eye, jnp.ones((d_out, 1), jnp.float32))
    seg_b = jnp.kron(eye, jnp.ones((1, d_out), jnp.float32))
    b1t = jnp.tile(b1, _GRP).reshape(1, vw)
    b2t = jnp.tile(b2, _GRP).reshape(1, ow)

    xwv = pl.pallas_call(
        _tc_xw_body,
        out_shape=jax.ShapeDtypeStruct((nv, vw), jnp.float32),
    )(x.reshape(nv, _GRP * d_in), w1b)

    degp = _make_deg(k_slices)(ei3, zeros_d, ones_d)

    y1v, disb = pl.pallas_call(
        _tc_scale1_body,
        out_shape=(
            jax.ShapeDtypeStruct((nv, vw), jnp.float32),
            jax.ShapeDtypeStruct((accv, vw), jnp.float32),
        ),
    )(xwv, degp.reshape(_NC, accv, vw))

    p1 = _make_agg(d_hid, k_slices, n)(y1v.reshape(n, d_hid), ei3, zeros_f)

    y2v = pl.pallas_call(
        _tc_mid_body,
        out_shape=jax.ShapeDtypeStruct((nv, vw), jnp.float32),
    )(p1.reshape(_NC, accv, vw), y1v, disb, b1t)

    p2 = _make_agg(d_hid, k_slices, n)(y2v.reshape(n, d_hid), ei3, zeros_f)

    outv = pl.pallas_call(
        _tc_final_body,
        out_shape=jax.ShapeDtypeStruct((nv, ow), jnp.float32),
    )(p2.reshape(_NC, accv, vw), y2v, disb, b2t, w2b, seg_s, seg_b)
    return outv.reshape(n, d_out)
